# Initial kernel scaffold; baseline (speedup 1.0000x reference)
import jax
import jax.numpy as jnp
from jax.experimental import pallas as pl


def kernel(isp_per_win, expert_indices, num_experts):
    raise NotImplementedError("write your pallas kernel here")


# speedup vs baseline: 1.5607x; 1.5607x over previous
import functools

import jax
import jax.numpy as jnp
from jax import lax
from jax.experimental import pallas as pl
from jax.experimental.pallas import tpu as pltpu
from jax.experimental.pallas import tpu_sc as plsc

NUM_CORES = 2
NUM_SUBCORES = 16
LANES = 16

WINDOWS = 4096
TOPK = 2
D = 1024
EXPERTS = 8
ROWS = WINDOWS * TOPK
OUT_ROWS = EXPERTS * WINDOWS

WIN_PER_TILE = WINDOWS // (NUM_CORES * NUM_SUBCORES)
ROWS_PER_TILE = WIN_PER_TILE * TOPK
CHUNK = 32
N_CHUNKS = ROWS_PER_TILE // CHUNK
ZROWS = 32
ZSTORES_PER_EXPERT = WIN_PER_TILE // ZROWS


def _dispatch_body(x_hbm, idx_hbm, out_hbm,
                   zslab, xbuf0, xbuf1, idx0, idx1, dst0, dst1,
                   zsem, gsem0, gsem1, ssem0, ssem1):
    c = lax.axis_index("c")
    s = lax.axis_index("s")
    w0 = (c * NUM_SUBCORES + s) * WIN_PER_TILE
    row0 = w0 * TOPK

    zero16 = jnp.zeros((LANES,), jnp.float32)

    @pl.loop(0, ZROWS)
    def _zrow(j):
        @pl.loop(0, D // LANES)
        def _zseg(i):
            zslab[j, pl.ds(i * LANES, LANES)] = zero16

    zhandles = []
    for e in range(EXPERTS):
        base = e * WINDOWS + w0
        for b in range(ZSTORES_PER_EXPERT):
            zhandles.append(
                pltpu.async_copy(
                    zslab, out_hbm.at[pl.ds(base + b * ZROWS, ZROWS)], zsem))
    for h in zhandles:
        h.wait()

    plsc.subcore_barrier()

    bufs = (xbuf0, xbuf1)
    idxs = (idx0, idx1)
    dsts = (dst0, dst1)
    gsems = (gsem0, gsem1)
    ssems = (ssem0, ssem1)
    scat = [None, None]
    for k in range(N_CHUNKS):
        p = k % 2
        rbase = row0 + k * CHUNK
        if scat[p] is not None:
            scat[p].wait()
        pltpu.sync_copy(idx_hbm.at[pl.ds(rbase, CHUNK)], idxs[p])
        g = pltpu.async_copy(x_hbm.at[pl.ds(rbase, CHUNK)], bufs[p], gsems[p])
        lane = lax.iota(jnp.int32, 16)
        for i in range(CHUNK // LANES):
            r = rbase + i * LANES + lane
            e = idxs[p][pl.ds(i * LANES, LANES)]
            dsts[p][pl.ds(i * LANES, LANES)] = e * WINDOWS + (r >> 1)
        g.wait()
        scat[p] = pltpu.async_copy(bufs[p], out_hbm.at[dsts[p]], ssems[p])
    for h in scat:
        if h is not None:
            h.wait()


_dispatch = functools.partial(
    pl.kernel,
    out_type=jax.ShapeDtypeStruct((OUT_ROWS, D), jnp.float32),
    mesh=plsc.VectorSubcoreMesh(
        core_axis_name="c", subcore_axis_name="s",
        num_cores=NUM_CORES, num_subcores=NUM_SUBCORES),
    scratch_types=[
        pltpu.VMEM((ZROWS, D), jnp.float32),
        pltpu.VMEM((CHUNK, D), jnp.float32),
        pltpu.VMEM((CHUNK, D), jnp.float32),
        pltpu.VMEM((CHUNK,), jnp.int32),
        pltpu.VMEM((CHUNK,), jnp.int32),
        pltpu.VMEM((CHUNK,), jnp.int32),
        pltpu.VMEM((CHUNK,), jnp.int32),
        pltpu.SemaphoreType.DMA,
        pltpu.SemaphoreType.DMA,
        pltpu.SemaphoreType.DMA,
        pltpu.SemaphoreType.DMA,
        pltpu.SemaphoreType.DMA,
    ],
)(_dispatch_body)


def kernel(isp_per_win, expert_indices, num_experts):
    batches, windows, k, embed_dim = isp_per_win.shape
    num_windows = batches * windows
    x = isp_per_win.reshape(num_windows * k, embed_dim)
    idx = expert_indices.reshape(-1)
    out = _dispatch(x, idx)
    return out.reshape(EXPERTS, num_windows, embed_dim)
